# baseline (speedup 1.0000x reference)
import functools

import jax
import jax.numpy as jnp
from jax import lax
from jax.experimental import pallas as pl
from jax.experimental.pallas import tpu as pltpu
from jax.experimental.pallas import tpu_sc as plsc

N = 10000
E = 320000
ENC = 128
EDGE = 16
LAT = 128

NUM_TILES = 16
EPT = E // NUM_TILES
C = 32
NCHUNK = EPT // C
N_PAD = 10240
TROWS = N_PAD // NUM_TILES
WB = 80


def _table_body(z_ref, w_ref, out_ref):
    out_ref[0] = jnp.dot(z_ref[...], w_ref[0],
                         preferred_element_type=jnp.float32)


def _node_table(z, w_stack):
    return pl.pallas_call(
        _table_body,
        grid=(4,),
        in_specs=[
            pl.BlockSpec((N, ENC), lambda j: (0, 0)),
            pl.BlockSpec((1, ENC, LAT), lambda j: (j, 0, 0)),
        ],
        out_specs=pl.BlockSpec((1, N, LAT), lambda j: (j, 0, 0)),
        out_shape=jax.ShapeDtypeStruct((4, N, LAT), jnp.float32),
    )(z, w_stack)


_BE = 8000


def _eproj_body(e_ref, w_ref, b_ref, out_ref):
    out_ref[0] = jnp.dot(e_ref[...], w_ref[0],
                         preferred_element_type=jnp.float32) + b_ref[0]


def _edge_proj(e_feat, we_stack, bm_stack):
    return pl.pallas_call(
        _eproj_body,
        grid=(2, E // _BE),
        in_specs=[
            pl.BlockSpec((_BE, EDGE), lambda p, i: (i, 0)),
            pl.BlockSpec((1, EDGE, LAT), lambda p, i: (p, 0, 0)),
            pl.BlockSpec((1, 1, LAT), lambda p, i: (p, 0, 0)),
        ],
        out_specs=pl.BlockSpec((1, _BE, LAT), lambda p, i: (p, i, 0)),
        out_shape=jax.ShapeDtypeStruct((2, E, LAT), jnp.float32),
    )(e_feat, we_stack, bm_stack)


def _sc_edge_body(t_hbm, ep_hbm, src_hbm, dst_hbm, out_hbm,
                  src_v0, dst_v0, gs_v0, gd_v0,
                  src_v1, dst_v1, gs_v1, gd_v1, sdst_v0, sdst_v1,
                  s_rows0, d_rows0, e_rows0, s_rows1, d_rows1, e_rows1,
                  agg_sh, sem_i0, sem_i1, sem_g0, sem_g1, sem_s0, sem_s1):
    c = lax.axis_index("c")
    s = lax.axis_index("s")

    src_v = (src_v0, src_v1)
    dst_v = (dst_v0, dst_v1)
    gs_v = (gs_v0, gs_v1)
    gd_v = (gd_v0, gd_v1)
    s_rows = (s_rows0, s_rows1)
    d_rows = (d_rows0, d_rows1)
    e_rows = (e_rows0, e_rows1)
    sdst_v = (sdst_v0, sdst_v1)
    sem_i = (sem_i0, sem_i1)
    sem_g = (sem_g0, sem_g1)
    sem_s = (sem_s0, sem_s1)

    def _wait_scatter(b):
        pltpu.make_async_copy(s_rows[b], agg_sh.at[sdst_v[b]],
                              sem_s[b]).wait()

    base0 = s * EPT
    soff = c * N
    doff = (2 + c) * N

    def _start_idx(i, b):
        pltpu.async_copy(src_hbm.at[pl.ds(base0 + i * C, C)], src_v[b],
                         sem_i[b])
        pltpu.async_copy(dst_hbm.at[pl.ds(base0 + i * C, C)], dst_v[b],
                         sem_i[b])

    def _wait_idx(b):
        pltpu.make_async_copy(src_hbm.at[pl.ds(0, C)], src_v[b],
                              sem_i[b]).wait()
        pltpu.make_async_copy(dst_hbm.at[pl.ds(0, C)], dst_v[b],
                              sem_i[b]).wait()

    def _mkidx(b):
        for k in range(C // 16):
            sl = pl.ds(k * 16, 16)
            gs_v[b][sl] = src_v[b][sl] + soff
            gd_v[b][sl] = dst_v[b][sl] + doff
            sdst_v[b][sl] = dst_v[b][sl]

    def _start_gathers(i, b):
        pltpu.async_copy(t_hbm.at[gs_v[b]], s_rows[b], sem_g[b])
        pltpu.async_copy(t_hbm.at[gd_v[b]], d_rows[b], sem_g[b])
        pltpu.async_copy(ep_hbm.at[pl.ds(c * E + base0 + i * C, C)],
                         e_rows[b], sem_g[b])

    def _wait_gathers(b):
        pltpu.make_async_copy(t_hbm.at[gs_v[b]], s_rows[b], sem_g[b]).wait()
        pltpu.make_async_copy(t_hbm.at[gd_v[b]], d_rows[b], sem_g[b]).wait()
        pltpu.make_async_copy(ep_hbm.at[pl.ds(0, C)], e_rows[b],
                              sem_g[b]).wait()

    _start_idx(0, 0)
    _start_idx(1, 1)

    def _zb(k, carry):
        s_rows0[k // 8, pl.ds((k % 8) * 16, 16)] = jnp.zeros((16,),
                                                             jnp.float32)
        return carry

    lax.fori_loop(0, C * 8, _zb, None)

    def _zcp(k, carry):
        pltpu.sync_copy(s_rows0, agg_sh.at[pl.ds(s * TROWS + k * C, C)])
        return carry

    lax.fori_loop(0, TROWS // C, _zcp, None)

    _wait_idx(0)
    _mkidx(0)
    _start_gathers(0, 0)
    plsc.subcore_barrier()

    def _chunk(i, carry):
        b = lax.rem(i, 2)

        def _iter(b_, nb_):
            @pl.when(i + 1 < NCHUNK)
            def _next():
                _wait_idx(nb_)

                @pl.when(i > 0)
                def _ws():
                    _wait_scatter(nb_)

                _mkidx(nb_)
                _start_gathers(i + 1, nb_)

            _wait_gathers(b_)

            def _relu(r, carry2):
                for j in range(8):
                    sl = pl.ds(j * 16, 16)
                    m = (s_rows[b_][r, sl] + d_rows[b_][r, sl]
                         + e_rows[b_][r, sl])
                    s_rows[b_][r, sl] = jnp.maximum(m, 0.0)
                return carry2

            lax.fori_loop(0, C, _relu, None)
            pltpu.async_copy(s_rows[b_], agg_sh.at[sdst_v[b_]], sem_s[b_],
                             add=True)

            @pl.when(i + 2 < NCHUNK)
            def _pf():
                _start_idx(i + 2, b_)

        @pl.when(b == 0)
        def _p0():
            _iter(0, 1)

        @pl.when(b == 1)
        def _p1():
            _iter(1, 0)

        return carry

    lax.fori_loop(0, NCHUNK, _chunk, None)
    _wait_scatter(0)
    _wait_scatter(1)
    plsc.subcore_barrier()

    r0 = s * TROWS
    ncp = jnp.minimum(TROWS // WB, (N - r0) // WB)

    def _ocp(k, carry):
        @pl.when(k < ncp)
        def _do():
            pltpu.sync_copy(agg_sh.at[pl.ds(r0 + k * WB, WB)],
                            out_hbm.at[pl.ds(c * N + r0 + k * WB, WB)])
        return carry

    lax.fori_loop(0, TROWS // WB, _ocp, None)


def _sc_edge(t_table, ep, src, dst):
    mesh = plsc.VectorSubcoreMesh(core_axis_name="c", subcore_axis_name="s")
    f = pl.kernel(
        _sc_edge_body,
        out_type=jax.ShapeDtypeStruct((2 * N, LAT), jnp.float32),
        mesh=mesh,
        scratch_types=(
            [pltpu.VMEM((C,), jnp.int32)] * 10
            + [pltpu.VMEM((C, LAT), jnp.float32)] * 6
            + [pltpu.VMEM_SHARED((N_PAD, LAT), jnp.float32)]
            + [pltpu.SemaphoreType.DMA] * 6
        ),
    )
    return f(t_table, ep, src, dst)


_BN = 2000


def _final_body(z_ref, a_ref, wu0_ref, bu0_ref, wu1_ref, bu1_ref, c_ref,
                out_ref):
    z = z_ref[...]
    o0 = (jnp.dot(z, wu0_ref[:ENC], preferred_element_type=jnp.float32)
          + jnp.dot(a_ref[0], wu0_ref[ENC:], preferred_element_type=jnp.float32)
          + bu0_ref[0])
    o1 = (jnp.dot(z, wu1_ref[:ENC], preferred_element_type=jnp.float32)
          + jnp.dot(a_ref[1], wu1_ref[ENC:], preferred_element_type=jnp.float32)
          + bu1_ref[0])
    out_ref[...] = c_ref[0, 0] * o0 + c_ref[0, 1] * o1


def _final(z, agg, Wu0, bu0, Wu1, bu1, coefs):
    return pl.pallas_call(
        _final_body,
        grid=(N // _BN,),
        in_specs=[
            pl.BlockSpec((_BN, ENC), lambda i: (i, 0)),
            pl.BlockSpec((2, _BN, LAT), lambda i: (0, i, 0)),
            pl.BlockSpec((ENC + LAT, ENC), lambda i: (0, 0)),
            pl.BlockSpec((1, ENC), lambda i: (0, 0)),
            pl.BlockSpec((ENC + LAT, ENC), lambda i: (0, 0)),
            pl.BlockSpec((1, ENC), lambda i: (0, 0)),
            pl.BlockSpec((1, 2), lambda i: (0, 0)),
        ],
        out_specs=pl.BlockSpec((_BN, ENC), lambda i: (i, 0)),
        out_shape=jax.ShapeDtypeStruct((N, ENC), jnp.float32),
    )(z, agg, Wu0, bu0, Wu1, bu1, coefs)


def kernel(z, e_feat, adj, Wm0, bm0, Wu0, bu0, coef0, Wm1, bm1, Wu1, bu1,
           coef1):
    src = adj[0].astype(jnp.int32)
    dst = adj[1].astype(jnp.int32)

    w_stack = jnp.stack([Wm0[:ENC], Wm1[:ENC],
                         Wm0[ENC:2 * ENC], Wm1[ENC:2 * ENC]])
    we_stack = jnp.stack([Wm0[2 * ENC:], Wm1[2 * ENC:]])
    bm_stack = jnp.stack([bm0, bm1]).reshape(2, 1, LAT)
    coefs = jnp.concatenate([coef0, coef1]).reshape(1, 2)

    t_table = _node_table(z, w_stack).reshape(4 * N, LAT)
    ep = _edge_proj(e_feat, we_stack, bm_stack).reshape(2 * E, LAT)
    agg = _sc_edge(t_table, ep, src, dst)
    return _final(z, agg.reshape(2, N, LAT), Wu0, bu0.reshape(1, ENC),
                  Wu1, bu1.reshape(1, ENC), coefs)


# speedup vs baseline: 1.0666x; 1.0666x over previous
import functools

import jax
import jax.numpy as jnp
from jax import lax
from jax.experimental import pallas as pl
from jax.experimental.pallas import tpu as pltpu
from jax.experimental.pallas import tpu_sc as plsc

N = 10000
E = 320000
ENC = 128
EDGE = 16
LAT = 128

NUM_TILES = 16
EPT = E // NUM_TILES
C = 40
NCHUNK = EPT // C
N_PAD = 10240
TROWS = N_PAD // NUM_TILES
WB = 80


def _table_body(z_ref, w_ref, out_ref):
    out_ref[0] = jnp.dot(z_ref[...], w_ref[0],
                         preferred_element_type=jnp.float32)


def _node_table(z, w_stack):
    return pl.pallas_call(
        _table_body,
        grid=(4,),
        in_specs=[
            pl.BlockSpec((N, ENC), lambda j: (0, 0)),
            pl.BlockSpec((1, ENC, LAT), lambda j: (j, 0, 0)),
        ],
        out_specs=pl.BlockSpec((1, N, LAT), lambda j: (j, 0, 0)),
        out_shape=jax.ShapeDtypeStruct((4, N, LAT), jnp.float32),
    )(z, w_stack)


_BE = 8000


def _eproj_body(e_ref, w_ref, b_ref, out_ref):
    out_ref[0] = jnp.dot(e_ref[...], w_ref[0],
                         preferred_element_type=jnp.float32) + b_ref[0]


def _edge_proj(e_feat, we_stack, bm_stack):
    return pl.pallas_call(
        _eproj_body,
        grid=(2, E // _BE),
        in_specs=[
            pl.BlockSpec((_BE, EDGE), lambda p, i: (i, 0)),
            pl.BlockSpec((1, EDGE, LAT), lambda p, i: (p, 0, 0)),
            pl.BlockSpec((1, 1, LAT), lambda p, i: (p, 0, 0)),
        ],
        out_specs=pl.BlockSpec((1, _BE, LAT), lambda p, i: (p, i, 0)),
        out_shape=jax.ShapeDtypeStruct((2, E, LAT), jnp.float32),
    )(e_feat, we_stack, bm_stack)


def _sc_edge_body(t_hbm, ep_hbm, src_hbm, dst_hbm, out_hbm,
                  src_v0, dst_v0, gs_v0, gd_v0,
                  src_v1, dst_v1, gs_v1, gd_v1, sdst_v0, sdst_v1,
                  s_rows0, d_rows0, e_rows0, s_rows1, d_rows1, e_rows1,
                  agg_sh, sem_i0, sem_i1, sem_g0, sem_g1, sem_s0, sem_s1):
    c = lax.axis_index("c")
    s = lax.axis_index("s")

    src_v = (src_v0, src_v1)
    dst_v = (dst_v0, dst_v1)
    gs_v = (gs_v0, gs_v1)
    gd_v = (gd_v0, gd_v1)
    s_rows = (s_rows0, s_rows1)
    d_rows = (d_rows0, d_rows1)
    e_rows = (e_rows0, e_rows1)
    sdst_v = (sdst_v0, sdst_v1)
    sem_i = (sem_i0, sem_i1)
    sem_g = (sem_g0, sem_g1)
    sem_s = (sem_s0, sem_s1)

    def _wait_scatter(b):
        pltpu.make_async_copy(s_rows[b], agg_sh.at[sdst_v[b]],
                              sem_s[b]).wait()

    base0 = s * EPT
    soff = c * N
    doff = (2 + c) * N

    def _start_idx(i, b):
        pltpu.async_copy(src_hbm.at[pl.ds(base0 + i * C, C)], src_v[b],
                         sem_i[b])
        pltpu.async_copy(dst_hbm.at[pl.ds(base0 + i * C, C)], dst_v[b],
                         sem_i[b])

    def _wait_idx(b):
        pltpu.make_async_copy(src_hbm.at[pl.ds(0, C)], src_v[b],
                              sem_i[b]).wait()
        pltpu.make_async_copy(dst_hbm.at[pl.ds(0, C)], dst_v[b],
                              sem_i[b]).wait()

    def _mkidx(b):
        for off in (0, 16, C - 16):
            sl = pl.ds(off, 16)
            gs_v[b][sl] = src_v[b][sl] + soff
            gd_v[b][sl] = dst_v[b][sl] + doff
            sdst_v[b][sl] = dst_v[b][sl]

    def _start_gathers(i, b):
        pltpu.async_copy(t_hbm.at[gs_v[b]], s_rows[b], sem_g[b])
        pltpu.async_copy(t_hbm.at[gd_v[b]], d_rows[b], sem_g[b])
        pltpu.async_copy(ep_hbm.at[pl.ds(c * E + base0 + i * C, C)],
                         e_rows[b], sem_g[b])

    def _wait_gathers(b):
        pltpu.make_async_copy(t_hbm.at[gs_v[b]], s_rows[b], sem_g[b]).wait()
        pltpu.make_async_copy(t_hbm.at[gd_v[b]], d_rows[b], sem_g[b]).wait()
        pltpu.make_async_copy(ep_hbm.at[pl.ds(0, C)], e_rows[b],
                              sem_g[b]).wait()

    _start_idx(0, 0)
    _start_idx(1, 1)

    def _zb(k, carry):
        s_rows0[k // 8, pl.ds((k % 8) * 16, 16)] = jnp.zeros((16,),
                                                             jnp.float32)
        return carry

    lax.fori_loop(0, C * 8, _zb, None)

    def _zcp(k, carry):
        pltpu.sync_copy(s_rows0, agg_sh.at[pl.ds(s * TROWS + k * C, C)])
        return carry

    lax.fori_loop(0, TROWS // C, _zcp, None)

    _wait_idx(0)
    _mkidx(0)
    _start_gathers(0, 0)
    plsc.subcore_barrier()

    def _chunk(i, carry):
        b = lax.rem(i, 2)

        def _iter(b_, nb_):
            @pl.when(i + 1 < NCHUNK)
            def _next():
                _wait_idx(nb_)

                @pl.when(i > 0)
                def _ws():
                    _wait_scatter(nb_)

                _mkidx(nb_)
                _start_gathers(i + 1, nb_)

            _wait_gathers(b_)

            def _relu(r, carry2):
                for j in range(8):
                    sl = pl.ds(j * 16, 16)
                    m = (s_rows[b_][r, sl] + d_rows[b_][r, sl]
                         + e_rows[b_][r, sl])
                    s_rows[b_][r, sl] = jnp.maximum(m, 0.0)
                return carry2

            lax.fori_loop(0, C, _relu, None)
            pltpu.async_copy(s_rows[b_], agg_sh.at[sdst_v[b_]], sem_s[b_],
                             add=True)

            @pl.when(i + 2 < NCHUNK)
            def _pf():
                _start_idx(i + 2, b_)

        @pl.when(b == 0)
        def _p0():
            _iter(0, 1)

        @pl.when(b == 1)
        def _p1():
            _iter(1, 0)

        return carry

    lax.fori_loop(0, NCHUNK, _chunk, None)
    _wait_scatter(0)
    _wait_scatter(1)
    plsc.subcore_barrier()

    r0 = s * TROWS
    ncp = jnp.minimum(TROWS // WB, (N - r0) // WB)

    def _ocp(k, carry):
        @pl.when(k < ncp)
        def _do():
            pltpu.sync_copy(agg_sh.at[pl.ds(r0 + k * WB, WB)],
                            out_hbm.at[pl.ds(c * N + r0 + k * WB, WB)])
        return carry

    lax.fori_loop(0, TROWS // WB, _ocp, None)


def _sc_edge(t_table, ep, src, dst):
    mesh = plsc.VectorSubcoreMesh(core_axis_name="c", subcore_axis_name="s")
    f = pl.kernel(
        _sc_edge_body,
        out_type=jax.ShapeDtypeStruct((2 * N, LAT), jnp.float32),
        mesh=mesh,
        scratch_types=(
            [pltpu.VMEM((C,), jnp.int32)] * 10
            + [pltpu.VMEM((C, LAT), jnp.float32)] * 6
            + [pltpu.VMEM_SHARED((N_PAD, LAT), jnp.float32)]
            + [pltpu.SemaphoreType.DMA] * 6
        ),
    )
    return f(t_table, ep, src, dst)


_BN = 2000


def _final_body(z_ref, a_ref, wu0_ref, bu0_ref, wu1_ref, bu1_ref, c_ref,
                out_ref):
    z = z_ref[...]
    o0 = (jnp.dot(z, wu0_ref[:ENC], preferred_element_type=jnp.float32)
          + jnp.dot(a_ref[0], wu0_ref[ENC:], preferred_element_type=jnp.float32)
          + bu0_ref[0])
    o1 = (jnp.dot(z, wu1_ref[:ENC], preferred_element_type=jnp.float32)
          + jnp.dot(a_ref[1], wu1_ref[ENC:], preferred_element_type=jnp.float32)
          + bu1_ref[0])
    out_ref[...] = c_ref[0, 0] * o0 + c_ref[0, 1] * o1


def _final(z, agg, Wu0, bu0, Wu1, bu1, coefs):
    return pl.pallas_call(
        _final_body,
        grid=(N // _BN,),
        in_specs=[
            pl.BlockSpec((_BN, ENC), lambda i: (i, 0)),
            pl.BlockSpec((2, _BN, LAT), lambda i: (0, i, 0)),
            pl.BlockSpec((ENC + LAT, ENC), lambda i: (0, 0)),
            pl.BlockSpec((1, ENC), lambda i: (0, 0)),
            pl.BlockSpec((ENC + LAT, ENC), lambda i: (0, 0)),
            pl.BlockSpec((1, ENC), lambda i: (0, 0)),
            pl.BlockSpec((1, 2), lambda i: (0, 0)),
        ],
        out_specs=pl.BlockSpec((_BN, ENC), lambda i: (i, 0)),
        out_shape=jax.ShapeDtypeStruct((N, ENC), jnp.float32),
    )(z, agg, Wu0, bu0, Wu1, bu1, coefs)


def kernel(z, e_feat, adj, Wm0, bm0, Wu0, bu0, coef0, Wm1, bm1, Wu1, bu1,
           coef1):
    src = adj[0].astype(jnp.int32)
    dst = adj[1].astype(jnp.int32)

    w_stack = jnp.stack([Wm0[:ENC], Wm1[:ENC],
                         Wm0[ENC:2 * ENC], Wm1[ENC:2 * ENC]])
    we_stack = jnp.stack([Wm0[2 * ENC:], Wm1[2 * ENC:]])
    bm_stack = jnp.stack([bm0, bm1]).reshape(2, 1, LAT)
    coefs = jnp.concatenate([coef0, coef1]).reshape(1, 2)

    t_table = _node_table(z, w_stack).reshape(4 * N, LAT)
    ep = _edge_proj(e_feat, we_stack, bm_stack).reshape(2 * E, LAT)
    agg = _sc_edge(t_table, ep, src, dst)
    return _final(z, agg.reshape(2, N, LAT), Wu0, bu0.reshape(1, ENC),
                  Wu1, bu1.reshape(1, ENC), coefs)
